# baseline (speedup 1.0000x reference)
import jax
import jax.numpy as jnp
from jax.experimental import pallas as pl
from jax.experimental.pallas import tpu as pltpu

_CB = 64


def _shift_kernel(idx_ref, x_ref, o_ref):
    T = x_ref.shape[2]
    o_ref[0] = x_ref[...]
    o_ref[1, :, :, : T - 1] = x_ref[:, :, 1:]
    o_ref[1, :, :, T - 1 :] = x_ref[:, :, T - 1 :]
    for n in range(idx_ref.shape[0]):
        i = idx_ref[n]
        o_ref[1, :, :, pl.ds(i, 1)] = x_ref[:, :, pl.ds(i, 1)]


def kernel(x, t_length):
    N, C, T, H, W = x.shape
    HW = H * W
    idx = jnp.mod(t_length.astype(jnp.int32) - 1, T)
    xr = x.reshape(N, C, T, HW)

    def in_map(n, c, iref):
        return (n, c, 0, 0)

    def out_map(n, c, iref):
        return (0, n, c, 0, 0)

    out = pl.pallas_call(
        _shift_kernel,
        grid_spec=pltpu.PrefetchScalarGridSpec(
            num_scalar_prefetch=1,
            grid=(N, C // _CB),
            in_specs=[pl.BlockSpec((1, _CB, T, HW), in_map)],
            out_specs=pl.BlockSpec((2, 1, _CB, T, HW), out_map),
        ),
        out_shape=jax.ShapeDtypeStruct((2, N, C, T, HW), x.dtype),
    )(idx, xr)
    return out.reshape(2, N, C, T, H, W)


# speedup vs baseline: 1.0206x; 1.0206x over previous
import jax
import jax.numpy as jnp
from jax.experimental import pallas as pl
from jax.experimental.pallas import tpu as pltpu

_CB = 256


def _shift_kernel(idx_ref, x_ref, o_ref):
    T = x_ref.shape[2]
    o_ref[0] = x_ref[...]
    o_ref[1, :, :, : T - 1] = x_ref[:, :, 1:]
    o_ref[1, :, :, T - 1 :] = x_ref[:, :, T - 1 :]
    for n in range(idx_ref.shape[0]):
        i = idx_ref[n]
        o_ref[1, :, :, pl.ds(i, 1)] = x_ref[:, :, pl.ds(i, 1)]


def kernel(x, t_length):
    N, C, T, H, W = x.shape
    HW = H * W
    idx = jnp.mod(t_length.astype(jnp.int32) - 1, T)
    xr = x.reshape(N, C, T, HW)

    def in_map(n, c, iref):
        return (n, c, 0, 0)

    def out_map(n, c, iref):
        return (0, n, c, 0, 0)

    out = pl.pallas_call(
        _shift_kernel,
        grid_spec=pltpu.PrefetchScalarGridSpec(
            num_scalar_prefetch=1,
            grid=(N, C // _CB),
            in_specs=[pl.BlockSpec((1, _CB, T, HW), in_map)],
            out_specs=pl.BlockSpec((2, 1, _CB, T, HW), out_map),
        ),
        out_shape=jax.ShapeDtypeStruct((2, N, C, T, HW), x.dtype),
    )(idx, xr)
    return out.reshape(2, N, C, T, H, W)
